# baseline (speedup 1.0000x reference)
import functools

import jax
import jax.numpy as jnp
from jax import lax
from jax.experimental import pallas as pl
from jax.experimental.pallas import tpu as pltpu
from jax.experimental.pallas import tpu_sc as plsc

_N = 20000
_D = 768
_H1 = 8
_E = 100000
_E_TOT = _E + _N
_E_PAD = 122880
_BN = 200
_N_PAD = _N

_NSC = 2
_NT = 16
_B = 128
_EPT = _E_PAD // _NT
_NB1 = _EPT // _B
_EPT2 = _E_PAD // (_NSC * _NT)
_NB2 = _EPT2 // _B
_RPT = _N // _NT
_ZROWS = 125


def _mm1_body(x_ref, w_ref, wa_ref, h_ref, ta_ref):
    h = jnp.dot(x_ref[...], w_ref[...], preferred_element_type=jnp.float32)
    h_ref[...] = h
    ta_ref[...] = jnp.dot(h, wa_ref[...], preferred_element_type=jnp.float32)


def _mm2_body(x_ref, b_ref, w_ref, wa_ref, h_ref, ta_ref):
    a = x_ref[...] + b_ref[...]
    a = jnp.where(a > 0, a, jnp.exp(jnp.minimum(a, 0.0)) - 1.0)
    h = jnp.dot(a, w_ref[...], preferred_element_type=jnp.float32)
    h_ref[...] = h
    ta_ref[...] = jnp.dot(h, wa_ref[...], preferred_element_type=jnp.float32)


def _matmul_alpha(x_pad, w, wa, *, bias=None):
    grid = (x_pad.shape[0] // _BN,)
    if bias is None:
        body = _mm1_body
        in_specs = [
            pl.BlockSpec((_BN, _D), lambda i: (i, 0)),
            pl.BlockSpec((_D, _D), lambda i: (0, 0)),
            pl.BlockSpec((_D, 128), lambda i: (0, 0)),
        ]
        args = (x_pad, w, wa)
    else:
        body = _mm2_body
        in_specs = [
            pl.BlockSpec((_BN, _D), lambda i: (i, 0)),
            pl.BlockSpec((1, _D), lambda i: (0, 0)),
            pl.BlockSpec((_D, _D), lambda i: (0, 0)),
            pl.BlockSpec((_D, 128), lambda i: (0, 0)),
        ]
        args = (x_pad, bias.reshape(1, _D), w, wa)
    h, ta = pl.pallas_call(
        body,
        grid=grid,
        in_specs=in_specs,
        out_specs=[
            pl.BlockSpec((_BN, _D), lambda i: (i, 0)),
            pl.BlockSpec((_BN, 128), lambda i: (i, 0)),
        ],
        out_shape=[
            jax.ShapeDtypeStruct((x_pad.shape[0], _D), jnp.float32),
            jax.ShapeDtypeStruct((x_pad.shape[0], 128), jnp.float32),
        ],
    )(*args)
    return h, ta


def _attn_body(n_heads, src_hbm, dst_hbm, t_hbm, alpha_hbm,
               srcb0, dstb0, tsrc0, tdst0, eeb0,
               srcb1, dstb1, tsrc1, tdst1, eeb1,
               denb, zb, sl0, sl1, sg0, sg1, denom_sh):
    s = lax.axis_index("s")
    perm = (lax.iota(jnp.int32, 16) % 8) + 8

    def zrow(i, _):
        zb[i] = jnp.zeros((16,), jnp.float32)
        return 0

    lax.fori_loop(0, _ZROWS, zrow, 0)
    r0 = s * _RPT
    for k in range(_RPT // _ZROWS):
        pltpu.sync_copy(zb, denom_sh.at[pl.ds(r0 + k * _ZROWS, _ZROWS)])
    plsc.subcore_barrier()

    def lin_start(gb, sb, db, sem):
        pltpu.async_copy(src_hbm.at[pl.ds(gb, _B)], sb, sem)
        pltpu.async_copy(dst_hbm.at[pl.ds(gb, _B)], db, sem)

    def lin_wait(gb, sb, db, sem):
        pltpu.make_async_copy(src_hbm.at[pl.ds(gb, _B)], sb, sem).wait()
        pltpu.make_async_copy(dst_hbm.at[pl.ds(gb, _B)], db, sem).wait()

    def gat_start(sb, db, ts, td, sem):
        pltpu.async_copy(t_hbm.at[sb], ts, sem)
        pltpu.async_copy(t_hbm.at[db], td, sem)

    def gat_wait(sb, db, ts, td, sem):
        pltpu.make_async_copy(t_hbm.at[sb], ts, sem).wait()
        pltpu.make_async_copy(t_hbm.at[db], td, sem).wait()

    def edge_rows(gb, ts, td, out_ref, div_ref):
        def row(i, _):
            e = ts[i] + td[i].at[perm].get(mode="promise_in_bounds")
            e = jnp.where(e > 0.0, e, 0.2 * e)
            hm = jnp.where(lax.iota(jnp.int32, 16) < n_heads,
                           jnp.float32(1.0), jnp.float32(0.0))
            ee = jnp.exp(e) * hm
            ee = ee * jnp.where(gb + i < _E_TOT, 1.0, 0.0)
            if div_ref is None:
                out_ref[i] = ee
            else:
                out_ref[i] = ee / (div_ref[i] + 1e-30)
            return 0

        lax.fori_loop(0, _B, row, 0)

    base = s * _EPT
    lin_start(base, srcb0, dstb0, sl0)

    def pair1(k2, _):
        b0 = base + (2 * k2) * _B
        b1 = b0 + _B
        lin_start(b1, srcb1, dstb1, sl1)
        lin_wait(b0, srcb0, dstb0, sl0)
        gat_start(srcb0, dstb0, tsrc0, tdst0, sg0)
        lin_wait(b1, srcb1, dstb1, sl1)
        gat_start(srcb1, dstb1, tsrc1, tdst1, sg1)
        gat_wait(srcb0, dstb0, tsrc0, tdst0, sg0)
        edge_rows(b0, tsrc0, tdst0, eeb0, None)
        pltpu.sync_copy(eeb0, denom_sh.at[dstb0], add=True)

        @pl.when(2 * k2 + 2 < _NB1)
        def _():
            lin_start(b0 + 2 * _B, srcb0, dstb0, sl0)

        gat_wait(srcb1, dstb1, tsrc1, tdst1, sg1)
        edge_rows(b1, tsrc1, tdst1, eeb1, None)
        pltpu.sync_copy(eeb1, denom_sh.at[dstb1], add=True)
        return 0

    lax.fori_loop(0, _NB1 // 2, pair1, 0)
    plsc.subcore_barrier()

    c = lax.axis_index("c")
    base2 = c * (_E_PAD // 2) + s * _EPT2
    lin_start(base2, srcb0, dstb0, sl0)

    def pair2(k2, _):
        b0 = base2 + (2 * k2) * _B
        b1 = b0 + _B
        lin_start(b1, srcb1, dstb1, sl1)
        lin_wait(b0, srcb0, dstb0, sl0)
        gat_start(srcb0, dstb0, tsrc0, tdst0, sg0)
        lin_wait(b1, srcb1, dstb1, sl1)
        gat_start(srcb1, dstb1, tsrc1, tdst1, sg1)
        gat_wait(srcb0, dstb0, tsrc0, tdst0, sg0)
        pltpu.sync_copy(denom_sh.at[dstb0], denb)
        edge_rows(b0, tsrc0, tdst0, eeb0, denb)
        pltpu.sync_copy(eeb0, alpha_hbm.at[pl.ds(b0, _B)])

        @pl.when(2 * k2 + 2 < _NB2)
        def _():
            lin_start(b0 + 2 * _B, srcb0, dstb0, sl0)

        gat_wait(srcb1, dstb1, tsrc1, tdst1, sg1)
        pltpu.sync_copy(denom_sh.at[dstb1], denb)
        edge_rows(b1, tsrc1, tdst1, eeb1, denb)
        pltpu.sync_copy(eeb1, alpha_hbm.at[pl.ds(b1, _B)])
        return 0

    lax.fori_loop(0, _NB2 // 2, pair2, 0)


def _attn_kernel(n_heads):
    mesh = plsc.VectorSubcoreMesh(
        core_axis_name="c", subcore_axis_name="s",
        num_cores=_NSC, num_subcores=_NT)
    return pl.kernel(
        functools.partial(_attn_body, n_heads),
        out_type=jax.ShapeDtypeStruct((_E_PAD, 16), jnp.float32),
        mesh=mesh,
        compiler_params=pltpu.CompilerParams(use_tc_tiling_on_sc=False),
        scratch_types=(
            2 * [
                pltpu.VMEM((_B,), jnp.int32),
                pltpu.VMEM((_B,), jnp.int32),
                pltpu.VMEM((_B, 16), jnp.float32),
                pltpu.VMEM((_B, 16), jnp.float32),
                pltpu.VMEM((_B, 16), jnp.float32),
            ] + [
                pltpu.VMEM((_B, 16), jnp.float32),
                pltpu.VMEM((_ZROWS, 16), jnp.float32),
                pltpu.SemaphoreType.DMA,
                pltpu.SemaphoreType.DMA,
                pltpu.SemaphoreType.DMA,
                pltpu.SemaphoreType.DMA,
                pltpu.VMEM_SHARED((_N, 16), jnp.float32),
            ]),
    )


_CH = 48
_CPS = 8


_ND = 4


def _scatter_body(per_head, src_hbm, dst_hbm, alpha_hbm, htab_hbm, out_hbm,
                  *scr):
    bufs = [scr[6 * m:6 * m + 6] for m in range(_ND)]
    zb, drb = scr[6 * _ND], scr[6 * _ND + 1]
    sl = scr[6 * _ND + 2:6 * _ND + 2 + _ND]
    sg = scr[6 * _ND + 2 + _ND:6 * _ND + 2 + 2 * _ND]
    sc_ = scr[6 * _ND + 2 + 2 * _ND:6 * _ND + 2 + 3 * _ND]
    acc_sh = scr[-1]
    c = lax.axis_index("c")
    s = lax.axis_index("s")
    r0 = s * _RPT
    base = s * _EPT

    def zrow(i, _):
        for k in range(_CH // 16):
            zb[i, 16 * k:16 * (k + 1)] = jnp.zeros((16,), jnp.float32)
        return 0

    lax.fori_loop(0, _ZROWS, zrow, 0)

    def lin_start(gb, m):
        sb, db, _, abuf, _, _ = bufs[m]
        pltpu.async_copy(src_hbm.at[pl.ds(gb, _B)], sb, sl[m])
        pltpu.async_copy(dst_hbm.at[pl.ds(gb, _B)], db, sl[m])
        pltpu.async_copy(alpha_hbm.at[pl.ds(gb, _B)], abuf, sl[m])

    def lin_wait(gb, m):
        sb, db, _, abuf, _, _ = bufs[m]
        pltpu.make_async_copy(src_hbm.at[pl.ds(gb, _B)], sb, sl[m]).wait()
        pltpu.make_async_copy(dst_hbm.at[pl.ds(gb, _B)], db, sl[m]).wait()
        pltpu.make_async_copy(alpha_hbm.at[pl.ds(gb, _B)], abuf, sl[m]).wait()

    def gidx_compute(m, g):
        sb, _, gxb, _, _, _ = bufs[m]

        def addoff(i, _):
            gxb[pl.ds(i * 16, 16)] = sb[pl.ds(i * 16, 16)] * 16 + g
            return 0

        lax.fori_loop(0, _B // 16, addoff, 0)

    def scale(m, g):
        _, _, _, abuf, rb, _ = bufs[m]

        def row(i, _):
            lane = (jnp.full((16,), g // 2, jnp.int32) if per_head
                    else jnp.zeros((16,), jnp.int32))
            arow = abuf[i]
            aval = arow.at[lane].get(mode="promise_in_bounds")
            for k in range(_CH // 16):
                slc = pl.ds(16 * k, 16)
                rb[i, slc] = rb[i, slc] * aval
            return 0

        lax.fori_loop(0, _B, row, 0)

    def chunk(j, _):
        g = _CPS * c + j
        for k in range(_RPT // _ZROWS):
            pltpu.sync_copy(zb, acc_sh.at[pl.ds(r0 + k * _ZROWS, _ZROWS)])
        plsc.subcore_barrier()

        for m in range(_ND - 1):
            lin_start(base + m * _B, m)

        def grp(kk, _):
            q0 = kk * _ND
            lin_start(base + (q0 + _ND - 1) * _B, _ND - 1)
            for m in range(_ND):
                lin_wait(base + (q0 + m) * _B, m)
                gidx_compute(m, g)

                @pl.when(kk > 0)
                def _():
                    pltpu.make_async_copy(
                        bufs[m][4], acc_sh.at[bufs[m][5]], sc_[m]).wait()

                pltpu.async_copy(htab_hbm.at[bufs[m][2]], bufs[m][4], sg[m])
            for m in range(_ND):
                pltpu.make_async_copy(
                    htab_hbm.at[bufs[m][2]], bufs[m][4], sg[m]).wait()
                scale(m, g)

                def dcopy(i, _, m=m):
                    bufs[m][5][pl.ds(i * 16, 16)] = \
                        bufs[m][1][pl.ds(i * 16, 16)]
                    return 0

                lax.fori_loop(0, _B // 16, dcopy, 0)
                pltpu.make_async_copy(
                    bufs[m][4], acc_sh.at[bufs[m][5]], sc_[m]).start(add=True)
                if m < _ND - 1:
                    @pl.when(q0 + m + _ND < _NB1)
                    def _():
                        lin_start(base + (q0 + m + _ND) * _B, m)
            return 0

        lax.fori_loop(0, _NB1 // _ND, grp, 0)
        for m in range(_ND):
            pltpu.make_async_copy(
                bufs[m][4], acc_sh.at[bufs[m][5]], sc_[m]).wait()
        plsc.subcore_barrier()

        for k in range(_RPT // _ZROWS):
            rr = r0 + k * _ZROWS
            pltpu.sync_copy(acc_sh.at[pl.ds(rr, _ZROWS)], drb)
            pltpu.sync_copy(drb, out_hbm.at[pl.ds(rr, _ZROWS), g])
        return 0

    lax.fori_loop(0, _CPS, chunk, 0)


def _scatter_kernel(per_head):
    mesh = plsc.VectorSubcoreMesh(
        core_axis_name="c", subcore_axis_name="s",
        num_cores=_NSC, num_subcores=_NT)
    return pl.kernel(
        functools.partial(_scatter_body, per_head),
        out_type=jax.ShapeDtypeStruct((_N_PAD, 16, _CH), jnp.float32),
        mesh=mesh,
        compiler_params=pltpu.CompilerParams(use_tc_tiling_on_sc=False),
        scratch_types=(
            _ND * [
                pltpu.VMEM((_B,), jnp.int32),
                pltpu.VMEM((_B,), jnp.int32),
                pltpu.VMEM((_B,), jnp.int32),
                pltpu.VMEM((_B, 16), jnp.float32),
                pltpu.VMEM((_B, _CH), jnp.float32),
                pltpu.VMEM((_B,), jnp.int32),
            ] + [
                pltpu.VMEM((_ZROWS, _CH), jnp.float32),
                pltpu.VMEM((_ZROWS, _CH), jnp.float32),
            ] + 3 * _ND * [pltpu.SemaphoreType.DMA]
            + [pltpu.VMEM_SHARED((_N, _CH), jnp.float32)]),
    )


def _edge_phase(h_pad, ta_pad, src_pad, dst_pad, n_heads):
    t_tab = ta_pad[:_N, :16]
    htab = h_pad.reshape(_N_PAD * 16, _CH)
    alpha = _attn_kernel(n_heads)(src_pad, dst_pad, t_tab)
    out = _scatter_kernel(n_heads == 8)(src_pad, dst_pad, alpha, htab)
    return out.reshape(_N_PAD, _D)


def kernel(x, edge_index, W1, a1_src, a1_dst, b1, W2, a2_src, a2_dst, b2):
    loops = jnp.arange(_N, dtype=jnp.int32)
    zpad = jnp.zeros((_E_PAD - _E_TOT,), jnp.int32)
    src_pad = jnp.concatenate([edge_index[0].astype(jnp.int32), loops, zpad])
    dst_pad = jnp.concatenate([edge_index[1].astype(jnp.int32), loops, zpad])

    head_ids = jnp.repeat(jnp.arange(_H1), _D // _H1)
    rows = jnp.arange(_D)
    wa1 = jnp.zeros((_D, 128), jnp.float32)
    wa1 = wa1.at[rows, head_ids].set(a1_src.reshape(-1))
    wa1 = wa1.at[rows, head_ids + 8].set(a1_dst.reshape(-1))
    wa2 = jnp.zeros((_D, 128), jnp.float32)
    wa2 = wa2.at[:, 0].set(a2_src.reshape(-1))
    wa2 = wa2.at[:, 8].set(a2_dst.reshape(-1))

    h1, ta1 = _matmul_alpha(x, W1, wa1)
    out1 = _edge_phase(h1, ta1, src_pad, dst_pad, _H1)

    h2, ta2 = _matmul_alpha(out1, W2, wa2, bias=b1)
    out2 = _edge_phase(h2, ta2, src_pad, dst_pad, 1)

    out = out2[:_N] + b2
    return (out, out[-1, :][None, :])


# speedup vs baseline: 1.0017x; 1.0017x over previous
import functools

import jax
import jax.numpy as jnp
from jax import lax
from jax.experimental import pallas as pl
from jax.experimental.pallas import tpu as pltpu
from jax.experimental.pallas import tpu_sc as plsc

_N = 20000
_D = 768
_H1 = 8
_E = 100000
_E_TOT = _E + _N
_E_PAD = 122880
_BN = 200
_N_PAD = _N

_NSC = 2
_NT = 16
_B = 128
_EPT = _E_PAD // _NT
_NB1 = _EPT // _B
_EPT2 = _E_PAD // (_NSC * _NT)
_NB2 = _EPT2 // _B
_RPT = _N // _NT
_ZROWS = 125


def _mm1_body(x_ref, w_ref, wa_ref, h_ref, ta_ref):
    h = jnp.dot(x_ref[...], w_ref[...], preferred_element_type=jnp.float32)
    h_ref[...] = h
    ta_ref[...] = jnp.dot(h, wa_ref[...], preferred_element_type=jnp.float32)


def _mm2_body(x_ref, b_ref, w_ref, wa_ref, h_ref, ta_ref):
    a = x_ref[...] + b_ref[...]
    a = jnp.where(a > 0, a, jnp.exp(jnp.minimum(a, 0.0)) - 1.0)
    h = jnp.dot(a, w_ref[...], preferred_element_type=jnp.float32)
    h_ref[...] = h
    ta_ref[...] = jnp.dot(h, wa_ref[...], preferred_element_type=jnp.float32)


def _matmul_alpha(x_pad, w, wa, *, bias=None):
    grid = (x_pad.shape[0] // _BN,)
    if bias is None:
        body = _mm1_body
        in_specs = [
            pl.BlockSpec((_BN, _D), lambda i: (i, 0)),
            pl.BlockSpec((_D, _D), lambda i: (0, 0)),
            pl.BlockSpec((_D, 128), lambda i: (0, 0)),
        ]
        args = (x_pad, w, wa)
    else:
        body = _mm2_body
        in_specs = [
            pl.BlockSpec((_BN, _D), lambda i: (i, 0)),
            pl.BlockSpec((1, _D), lambda i: (0, 0)),
            pl.BlockSpec((_D, _D), lambda i: (0, 0)),
            pl.BlockSpec((_D, 128), lambda i: (0, 0)),
        ]
        args = (x_pad, bias.reshape(1, _D), w, wa)
    h, ta = pl.pallas_call(
        body,
        grid=grid,
        in_specs=in_specs,
        out_specs=[
            pl.BlockSpec((_BN, _D), lambda i: (i, 0)),
            pl.BlockSpec((_BN, 128), lambda i: (i, 0)),
        ],
        out_shape=[
            jax.ShapeDtypeStruct((x_pad.shape[0], _D), jnp.float32),
            jax.ShapeDtypeStruct((x_pad.shape[0], 128), jnp.float32),
        ],
    )(*args)
    return h, ta


def _attn_body(n_heads, src_hbm, dst_hbm, t_hbm, alpha_hbm,
               srcb0, dstb0, tsrc0, tdst0, eeb0,
               srcb1, dstb1, tsrc1, tdst1, eeb1,
               denb, zb, sl0, sl1, sg0, sg1, denom_sh):
    s = lax.axis_index("s")
    perm = (lax.iota(jnp.int32, 16) % 8) + 8

    def zrow(i, _):
        zb[i] = jnp.zeros((16,), jnp.float32)
        return 0

    lax.fori_loop(0, _ZROWS, zrow, 0)
    r0 = s * _RPT
    for k in range(_RPT // _ZROWS):
        pltpu.sync_copy(zb, denom_sh.at[pl.ds(r0 + k * _ZROWS, _ZROWS)])
    plsc.subcore_barrier()

    def lin_start(gb, sb, db, sem):
        pltpu.async_copy(src_hbm.at[pl.ds(gb, _B)], sb, sem)
        pltpu.async_copy(dst_hbm.at[pl.ds(gb, _B)], db, sem)

    def lin_wait(gb, sb, db, sem):
        pltpu.make_async_copy(src_hbm.at[pl.ds(gb, _B)], sb, sem).wait()
        pltpu.make_async_copy(dst_hbm.at[pl.ds(gb, _B)], db, sem).wait()

    def gat_start(sb, db, ts, td, sem):
        pltpu.async_copy(t_hbm.at[sb], ts, sem)
        pltpu.async_copy(t_hbm.at[db], td, sem)

    def gat_wait(sb, db, ts, td, sem):
        pltpu.make_async_copy(t_hbm.at[sb], ts, sem).wait()
        pltpu.make_async_copy(t_hbm.at[db], td, sem).wait()

    def edge_rows(gb, ts, td, out_ref, div_ref):
        def row(i, _):
            e = ts[i] + td[i].at[perm].get(mode="promise_in_bounds")
            e = jnp.where(e > 0.0, e, 0.2 * e)
            hm = jnp.where(lax.iota(jnp.int32, 16) < n_heads,
                           jnp.float32(1.0), jnp.float32(0.0))
            ee = jnp.exp(e) * hm
            ee = ee * jnp.where(gb + i < _E_TOT, 1.0, 0.0)
            if div_ref is None:
                out_ref[i] = ee
            else:
                out_ref[i] = ee / (div_ref[i] + 1e-30)
            return 0

        lax.fori_loop(0, _B, row, 0)

    base = s * _EPT
    lin_start(base, srcb0, dstb0, sl0)

    def pair1(k2, _):
        b0 = base + (2 * k2) * _B
        b1 = b0 + _B
        lin_start(b1, srcb1, dstb1, sl1)
        lin_wait(b0, srcb0, dstb0, sl0)
        gat_start(srcb0, dstb0, tsrc0, tdst0, sg0)
        lin_wait(b1, srcb1, dstb1, sl1)
        gat_start(srcb1, dstb1, tsrc1, tdst1, sg1)
        gat_wait(srcb0, dstb0, tsrc0, tdst0, sg0)
        edge_rows(b0, tsrc0, tdst0, eeb0, None)
        pltpu.sync_copy(eeb0, denom_sh.at[dstb0], add=True)

        @pl.when(2 * k2 + 2 < _NB1)
        def _():
            lin_start(b0 + 2 * _B, srcb0, dstb0, sl0)

        gat_wait(srcb1, dstb1, tsrc1, tdst1, sg1)
        edge_rows(b1, tsrc1, tdst1, eeb1, None)
        pltpu.sync_copy(eeb1, denom_sh.at[dstb1], add=True)
        return 0

    lax.fori_loop(0, _NB1 // 2, pair1, 0)
    plsc.subcore_barrier()

    c = lax.axis_index("c")
    base2 = c * (_E_PAD // 2) + s * _EPT2
    lin_start(base2, srcb0, dstb0, sl0)

    def pair2(k2, _):
        b0 = base2 + (2 * k2) * _B
        b1 = b0 + _B
        lin_start(b1, srcb1, dstb1, sl1)
        lin_wait(b0, srcb0, dstb0, sl0)
        gat_start(srcb0, dstb0, tsrc0, tdst0, sg0)
        lin_wait(b1, srcb1, dstb1, sl1)
        gat_start(srcb1, dstb1, tsrc1, tdst1, sg1)
        gat_wait(srcb0, dstb0, tsrc0, tdst0, sg0)
        pltpu.sync_copy(denom_sh.at[dstb0], denb)
        edge_rows(b0, tsrc0, tdst0, eeb0, denb)
        pltpu.sync_copy(eeb0, alpha_hbm.at[pl.ds(b0, _B)])

        @pl.when(2 * k2 + 2 < _NB2)
        def _():
            lin_start(b0 + 2 * _B, srcb0, dstb0, sl0)

        gat_wait(srcb1, dstb1, tsrc1, tdst1, sg1)
        pltpu.sync_copy(denom_sh.at[dstb1], denb)
        edge_rows(b1, tsrc1, tdst1, eeb1, denb)
        pltpu.sync_copy(eeb1, alpha_hbm.at[pl.ds(b1, _B)])
        return 0

    lax.fori_loop(0, _NB2 // 2, pair2, 0)


def _attn_kernel(n_heads):
    mesh = plsc.VectorSubcoreMesh(
        core_axis_name="c", subcore_axis_name="s",
        num_cores=_NSC, num_subcores=_NT)
    return pl.kernel(
        functools.partial(_attn_body, n_heads),
        out_type=jax.ShapeDtypeStruct((_E_PAD, 16), jnp.float32),
        mesh=mesh,
        compiler_params=pltpu.CompilerParams(use_tc_tiling_on_sc=False),
        scratch_types=(
            2 * [
                pltpu.VMEM((_B,), jnp.int32),
                pltpu.VMEM((_B,), jnp.int32),
                pltpu.VMEM((_B, 16), jnp.float32),
                pltpu.VMEM((_B, 16), jnp.float32),
                pltpu.VMEM((_B, 16), jnp.float32),
            ] + [
                pltpu.VMEM((_B, 16), jnp.float32),
                pltpu.VMEM((_ZROWS, 16), jnp.float32),
                pltpu.SemaphoreType.DMA,
                pltpu.SemaphoreType.DMA,
                pltpu.SemaphoreType.DMA,
                pltpu.SemaphoreType.DMA,
                pltpu.VMEM_SHARED((_N, 16), jnp.float32),
            ]),
    )


_CH = 48
_CPS = 8


_ND = 4


def _scatter_body(per_head, src_hbm, dst_hbm, alpha_hbm, htab_hbm, out_hbm,
                  *scr):
    bufs = [scr[6 * m:6 * m + 6] for m in range(_ND)]
    zb, drb = scr[6 * _ND], scr[6 * _ND + 1]
    sl = scr[6 * _ND + 2:6 * _ND + 2 + _ND]
    sg = scr[6 * _ND + 2 + _ND:6 * _ND + 2 + 2 * _ND]
    sc_ = scr[6 * _ND + 2 + 2 * _ND:6 * _ND + 2 + 3 * _ND]
    acc_sh = scr[-1]
    c = lax.axis_index("c")
    s = lax.axis_index("s")
    r0 = s * _RPT
    base = s * _EPT

    def zrow(i, _):
        for k in range(_CH // 16):
            zb[i, 16 * k:16 * (k + 1)] = jnp.zeros((16,), jnp.float32)
        return 0

    lax.fori_loop(0, _ZROWS, zrow, 0)

    def lin_start(gb, m):
        sb, db, _, abuf, _, _ = bufs[m]
        pltpu.async_copy(src_hbm.at[pl.ds(gb, _B)], sb, sl[m])
        pltpu.async_copy(dst_hbm.at[pl.ds(gb, _B)], db, sl[m])
        pltpu.async_copy(alpha_hbm.at[pl.ds(gb, _B)], abuf, sl[m])

    def lin_wait(gb, m):
        sb, db, _, abuf, _, _ = bufs[m]
        pltpu.make_async_copy(src_hbm.at[pl.ds(gb, _B)], sb, sl[m]).wait()
        pltpu.make_async_copy(dst_hbm.at[pl.ds(gb, _B)], db, sl[m]).wait()
        pltpu.make_async_copy(alpha_hbm.at[pl.ds(gb, _B)], abuf, sl[m]).wait()

    def gidx_compute(m, g):
        sb, _, gxb, _, _, _ = bufs[m]

        def addoff(i, _):
            gxb[pl.ds(i * 16, 16)] = sb[pl.ds(i * 16, 16)] * 16 + g
            return 0

        lax.fori_loop(0, _B // 16, addoff, 0)

    def scale(m, g):
        _, _, _, abuf, rb, _ = bufs[m]

        def row4(i4, _):
            for u in range(4):
                i = i4 * 4 + u
                lane = (jnp.full((16,), g // 2, jnp.int32) if per_head
                        else jnp.zeros((16,), jnp.int32))
                arow = abuf[i]
                aval = arow.at[lane].get(mode="promise_in_bounds")
                for k in range(_CH // 16):
                    slc = pl.ds(16 * k, 16)
                    rb[i, slc] = rb[i, slc] * aval
            return 0

        lax.fori_loop(0, _B // 4, row4, 0)

    def chunk(j, _):
        g = _CPS * c + j
        for k in range(_RPT // _ZROWS):
            pltpu.sync_copy(zb, acc_sh.at[pl.ds(r0 + k * _ZROWS, _ZROWS)])
        plsc.subcore_barrier()

        for m in range(_ND - 1):
            lin_start(base + m * _B, m)

        def grp(kk, _):
            q0 = kk * _ND
            lin_start(base + (q0 + _ND - 1) * _B, _ND - 1)
            for m in range(_ND):
                lin_wait(base + (q0 + m) * _B, m)
                gidx_compute(m, g)

                @pl.when(kk > 0)
                def _():
                    pltpu.make_async_copy(
                        bufs[m][4], acc_sh.at[bufs[m][5]], sc_[m]).wait()

                pltpu.async_copy(htab_hbm.at[bufs[m][2]], bufs[m][4], sg[m])
            for m in range(_ND):
                pltpu.make_async_copy(
                    htab_hbm.at[bufs[m][2]], bufs[m][4], sg[m]).wait()
                scale(m, g)

                def dcopy(i, _, m=m):
                    bufs[m][5][pl.ds(i * 16, 16)] = \
                        bufs[m][1][pl.ds(i * 16, 16)]
                    return 0

                lax.fori_loop(0, _B // 16, dcopy, 0)
                pltpu.make_async_copy(
                    bufs[m][4], acc_sh.at[bufs[m][5]], sc_[m]).start(add=True)
                if m < _ND - 1:
                    @pl.when(q0 + m + _ND < _NB1)
                    def _():
                        lin_start(base + (q0 + m + _ND) * _B, m)
            return 0

        lax.fori_loop(0, _NB1 // _ND, grp, 0)
        for m in range(_ND):
            pltpu.make_async_copy(
                bufs[m][4], acc_sh.at[bufs[m][5]], sc_[m]).wait()
        plsc.subcore_barrier()

        for k in range(_RPT // _ZROWS):
            rr = r0 + k * _ZROWS
            pltpu.sync_copy(acc_sh.at[pl.ds(rr, _ZROWS)], drb)
            pltpu.sync_copy(drb, out_hbm.at[pl.ds(rr, _ZROWS), g])
        return 0

    lax.fori_loop(0, _CPS, chunk, 0)


def _scatter_kernel(per_head):
    mesh = plsc.VectorSubcoreMesh(
        core_axis_name="c", subcore_axis_name="s",
        num_cores=_NSC, num_subcores=_NT)
    return pl.kernel(
        functools.partial(_scatter_body, per_head),
        out_type=jax.ShapeDtypeStruct((_N_PAD, 16, _CH), jnp.float32),
        mesh=mesh,
        compiler_params=pltpu.CompilerParams(use_tc_tiling_on_sc=False),
        scratch_types=(
            _ND * [
                pltpu.VMEM((_B,), jnp.int32),
                pltpu.VMEM((_B,), jnp.int32),
                pltpu.VMEM((_B,), jnp.int32),
                pltpu.VMEM((_B, 16), jnp.float32),
                pltpu.VMEM((_B, _CH), jnp.float32),
                pltpu.VMEM((_B,), jnp.int32),
            ] + [
                pltpu.VMEM((_ZROWS, _CH), jnp.float32),
                pltpu.VMEM((_ZROWS, _CH), jnp.float32),
            ] + 3 * _ND * [pltpu.SemaphoreType.DMA]
            + [pltpu.VMEM_SHARED((_N, _CH), jnp.float32)]),
    )


def _edge_phase(h_pad, ta_pad, src_pad, dst_pad, n_heads):
    t_tab = ta_pad[:_N, :16]
    htab = h_pad.reshape(_N_PAD * 16, _CH)
    alpha = _attn_kernel(n_heads)(src_pad, dst_pad, t_tab)
    out = _scatter_kernel(n_heads == 8)(src_pad, dst_pad, alpha, htab)
    return out.reshape(_N_PAD, _D)


def kernel(x, edge_index, W1, a1_src, a1_dst, b1, W2, a2_src, a2_dst, b2):
    loops = jnp.arange(_N, dtype=jnp.int32)
    zpad = jnp.zeros((_E_PAD - _E_TOT,), jnp.int32)
    src_pad = jnp.concatenate([edge_index[0].astype(jnp.int32), loops, zpad])
    dst_pad = jnp.concatenate([edge_index[1].astype(jnp.int32), loops, zpad])

    head_ids = jnp.repeat(jnp.arange(_H1), _D // _H1)
    rows = jnp.arange(_D)
    wa1 = jnp.zeros((_D, 128), jnp.float32)
    wa1 = wa1.at[rows, head_ids].set(a1_src.reshape(-1))
    wa1 = wa1.at[rows, head_ids + 8].set(a1_dst.reshape(-1))
    wa2 = jnp.zeros((_D, 128), jnp.float32)
    wa2 = wa2.at[:, 0].set(a2_src.reshape(-1))
    wa2 = wa2.at[:, 8].set(a2_dst.reshape(-1))

    h1, ta1 = _matmul_alpha(x, W1, wa1)
    out1 = _edge_phase(h1, ta1, src_pad, dst_pad, _H1)

    h2, ta2 = _matmul_alpha(out1, W2, wa2, bias=b1)
    out2 = _edge_phase(h2, ta2, src_pad, dst_pad, 1)

    out = out2[:_N] + b2
    return (out, out[-1, :][None, :])
